# Initial kernel scaffold; baseline (speedup 1.0000x reference)
import jax
import jax.numpy as jnp
from jax.experimental import pallas as pl


def kernel(mask_i):
    raise NotImplementedError("write your pallas kernel here")


# speedup vs baseline: 3.4417x; 3.4417x over previous
import functools

import jax
import jax.numpy as jnp
from jax import lax
from jax.experimental import pallas as pl
from jax.experimental.pallas import tpu as pltpu
from jax.experimental.pallas import tpu_sc as plsc

N = 2097152
NC = 2
NS = 16
NW = NC * NS
CHUNK = N // NW
LANES = 16
VECS = CHUNK // LANES

_mesh = plsc.VectorSubcoreMesh(core_axis_name="c", subcore_axis_name="s")
_params = pltpu.CompilerParams(needs_layout_passes=False)


def _wid():
    return lax.axis_index("c") * NS + lax.axis_index("s")


@functools.partial(
    pl.kernel,
    out_type=jax.ShapeDtypeStruct((NW * LANES,), jnp.float32),
    mesh=_mesh,
    compiler_params=_params,
    scratch_types=[
        pltpu.VMEM((CHUNK,), jnp.float32),
        pltpu.VMEM((LANES,), jnp.float32),
    ],
)
def _chunk_sums(x_hbm, out_hbm, buf, accv):
    wid = _wid()
    pltpu.sync_copy(x_hbm.at[pl.ds(wid * CHUNK, CHUNK)], buf)

    def body(i, acc):
        return acc + buf[pl.ds(i * LANES, LANES)]

    acc = lax.fori_loop(0, VECS, body, jnp.zeros((LANES,), jnp.float32))
    accv[...] = acc
    pltpu.sync_copy(accv, out_hbm.at[pl.ds(wid * LANES, LANES)])


@functools.partial(
    pl.kernel,
    out_type=jax.ShapeDtypeStruct((N,), jnp.float32),
    mesh=_mesh,
    compiler_params=_params,
    scratch_types=[
        pltpu.VMEM((CHUNK,), jnp.float32),
        pltpu.VMEM((NW * LANES,), jnp.float32),
    ],
)
def _scan_chunks(x_hbm, sums_hbm, out_hbm, buf, sums_v):
    wid = _wid()
    pltpu.sync_copy(x_hbm.at[pl.ds(wid * CHUNK, CHUNK)], buf)
    pltpu.sync_copy(sums_hbm, sums_v)

    def off_body(w, acc):
        v = sums_v[pl.ds(w * LANES, LANES)]
        keep = (w < wid).astype(jnp.float32)
        return acc + v * keep

    offv = lax.fori_loop(0, NW, off_body, jnp.zeros((LANES,), jnp.float32))
    carry0 = jnp.sum(offv) - 1.0

    def body(i, carry):
        v = buf[pl.ds(i * LANES, LANES)]
        s = jnp.cumsum(v)
        buf[pl.ds(i * LANES, LANES)] = s + carry
        return carry + jnp.sum(v)

    lax.fori_loop(0, VECS, body, carry0)
    pltpu.sync_copy(buf, out_hbm.at[pl.ds(wid * CHUNK, CHUNK)])


def kernel(mask_i):
    sums = _chunk_sums(mask_i)
    return _scan_chunks(mask_i, sums)


# speedup vs baseline: 7.1116x; 2.0663x over previous
import functools

import jax
import jax.numpy as jnp
from jax import lax
from jax.experimental import pallas as pl
from jax.experimental.pallas import tpu as pltpu
from jax.experimental.pallas import tpu_sc as plsc

N = 2097152
NC = 2
NS = 16
NW = NC * NS
CHUNK = N // NW
LANES = 16
VECS = CHUNK // LANES
_U = 8

_mesh = plsc.VectorSubcoreMesh(core_axis_name="c", subcore_axis_name="s")
_params = pltpu.CompilerParams(needs_layout_passes=False)


def _wid():
    return lax.axis_index("c") * NS + lax.axis_index("s")


@functools.partial(
    pl.kernel,
    out_type=jax.ShapeDtypeStruct((NW * LANES,), jnp.float32),
    mesh=_mesh,
    compiler_params=_params,
    scratch_types=[
        pltpu.VMEM((CHUNK,), jnp.float32),
        pltpu.VMEM((LANES,), jnp.float32),
    ],
)
def _chunk_sums(x_hbm, out_hbm, buf, accv):
    wid = _wid()
    pltpu.sync_copy(x_hbm.at[pl.ds(wid * CHUNK, CHUNK)], buf)

    def body(g, accs):
        base = g * (_U * LANES)
        accs = list(accs)
        for j in range(_U):
            accs[j % 4] = accs[j % 4] + buf[pl.ds(base + j * LANES, LANES)]
        return tuple(accs)

    z = jnp.zeros((LANES,), jnp.float32)
    a0, a1, a2, a3 = lax.fori_loop(0, VECS // _U, body, (z, z, z, z))
    accv[...] = (a0 + a1) + (a2 + a3)
    pltpu.sync_copy(accv, out_hbm.at[pl.ds(wid * LANES, LANES)])


@functools.partial(
    pl.kernel,
    out_type=jax.ShapeDtypeStruct((N,), jnp.float32),
    mesh=_mesh,
    compiler_params=_params,
    scratch_types=[
        pltpu.VMEM((CHUNK,), jnp.float32),
        pltpu.VMEM((NW * LANES,), jnp.float32),
    ],
)
def _scan_chunks(x_hbm, sums_hbm, out_hbm, buf, sums_v):
    wid = _wid()
    pltpu.sync_copy(x_hbm.at[pl.ds(wid * CHUNK, CHUNK)], buf)
    pltpu.sync_copy(sums_hbm, sums_v)

    def off_body(w, acc):
        v = sums_v[pl.ds(w * LANES, LANES)]
        keep = (w < wid).astype(jnp.float32)
        return acc + v * keep

    offv = lax.fori_loop(0, NW, off_body, jnp.zeros((LANES,), jnp.float32))
    carry0 = jnp.sum(offv) - 1.0

    def body(g, carry):
        base = g * (_U * LANES)
        ss = []
        ts = []
        for j in range(_U):
            v = buf[pl.ds(base + j * LANES, LANES)]
            s = jnp.cumsum(v)
            ss.append(s)
            ts.append(s[15])
        t01 = ts[0] + ts[1]
        t23 = ts[2] + ts[3]
        t45 = ts[4] + ts[5]
        t67 = ts[6] + ts[7]
        t03 = t01 + t23
        e = [None] * _U
        e[1] = ts[0]
        e[2] = t01
        e[3] = t01 + ts[2]
        e[4] = t03
        e[5] = t03 + ts[4]
        e[6] = t03 + t45
        e[7] = e[6] + ts[6]
        buf[pl.ds(base, LANES)] = ss[0] + carry
        for j in range(1, _U):
            buf[pl.ds(base + j * LANES, LANES)] = ss[j] + (carry + e[j])
        return carry + (t03 + t45 + t67)

    lax.fori_loop(0, VECS // _U, body, carry0)
    pltpu.sync_copy(buf, out_hbm.at[pl.ds(wid * CHUNK, CHUNK)])


def kernel(mask_i):
    sums = _chunk_sums(mask_i)
    return _scan_chunks(mask_i, sums)
